# baseline (speedup 1.0000x reference)
import functools

import jax
import jax.numpy as jnp
from jax import lax
from jax.experimental import pallas as pl
from jax.experimental.pallas import tpu as pltpu
from jax.experimental.pallas import tpu_sc as plsc

N = 10000
NPAD = 10240
H = 128
K = 128
BLK = 256
F32 = jnp.float32
DROWS = NPAD // 16

_mesh = plsc.VectorSubcoreMesh(core_axis_name="c", subcore_axis_name="s")
_sc_params = pltpu.CompilerParams(needs_layout_passes=False,
                                  use_tc_tiling_on_sc=False)


HH = H // 2
CH = 2048


def _make_sc_scatter(e_pad):
    ept = e_pad // 16
    assert ept % CH == 0 and CH % K == 0

    def body(src_h, dst_h, asrc_h, adst_h, h_h,
             num_h, den_h,
             srcv, dstv, asrc_v, adst_v, den_v, ex_v,
             rows0_v, rows1_v, srcs0_v, srcs1_v, dsts0_v, dsts1_v,
             sem0, sem1,
             sh_num, sh_den):
        rows_v = rows0_v
        dsts_v = dsts0_v
        sid = lax.axis_index("s")
        cid = lax.axis_index("c")
        e0 = sid * ept
        srows = NPAD // 16

        z16 = jnp.zeros((16,), F32)

        def zrow(r, _):
            for j in range(HH // 16):
                rows_v[r, pl.ds(16 * j, 16)] = z16
            return 0
        lax.fori_loop(0, K, zrow, 0)

        def zden(r, _):
            den_v[r] = z16
            return 0
        lax.fori_loop(0, DROWS, zden, 0)

        for b in range(srows // K):
            pltpu.sync_copy(rows_v, sh_num.at[pl.ds(sid * srows + b * K, K)])

        @pl.when(cid == 0)
        def _():
            pltpu.sync_copy(den_v.at[pl.ds(0, DROWS // 16)],
                            sh_den.at[pl.ds(sid * (DROWS // 16),
                                            DROWS // 16)])

        pltpu.sync_copy(asrc_h, asrc_v)
        pltpu.sync_copy(adst_h, adst_v)

        plsc.subcore_barrier()

        def chunk(c, _):
            pltpu.sync_copy(src_h.at[pl.ds(e0 + c * CH, CH)], srcv)
            pltpu.sync_copy(dst_h.at[pl.ds(e0 + c * CH, CH)], dstv)

            def grp(g):
                s = srcv[pl.ds(g * 16, 16)]
                d = dstv[pl.ds(g * 16, 16)]
                av = plsc.load_gather(asrc_v, [s])
                ad = plsc.load_gather(adst_v, [d])
                e = av + ad
                e = jnp.where(e >= 0, e, 0.2 * e)
                ex = jnp.exp(e)
                ex_v[pl.ds(g * 16, 16)] = ex

            def grp_den(g):
                s = srcv[pl.ds(g * 16, 16)]
                d = dstv[pl.ds(g * 16, 16)]
                av = plsc.load_gather(asrc_v, [s])
                ad = plsc.load_gather(adst_v, [d])
                e = av + ad
                e = jnp.where(e >= 0, e, 0.2 * e)
                ex = jnp.exp(e)
                ex_v[pl.ds(g * 16, 16)] = ex
                plsc.addupdate_scatter(
                    den_v, [lax.shift_right_logical(d, 4), d & 15], ex)

            @pl.when(cid == 0)
            def _():
                plsc.parallel_loop(0, CH // 16, 1, unroll=4)(grp_den)

            @pl.when(cid == 1)
            def _():
                plsc.parallel_loop(0, CH // 16, 1, unroll=4)(grp)

            def weight(buf, base):
                @plsc.parallel_loop(0, K, 1, unroll=4)
                def _(r):
                    w = jnp.full((16,), ex_v[pl.ds(base + r, 16)][0])
                    for j in range(HH // 16):
                        sl = pl.ds(16 * j, 16)
                        buf[r, sl] = buf[r, sl] * w

            def fpair(fp, _):
                base0 = fp * 2 * K
                base1 = base0 + K
                for j in range(K // 16):
                    srcs0_v[pl.ds(16 * j, 16)] = srcv[pl.ds(base0 + 16 * j, 16)]
                h0 = pltpu.async_copy(h_h.at[cid].at[srcs0_v], rows0_v, sem0)
                for j in range(K // 16):
                    srcs1_v[pl.ds(16 * j, 16)] = srcv[pl.ds(base1 + 16 * j, 16)]
                h1 = pltpu.async_copy(h_h.at[cid].at[srcs1_v], rows1_v, sem1)
                for j in range(K // 16):
                    dsts0_v[pl.ds(16 * j, 16)] = dstv[pl.ds(base0 + 16 * j, 16)]
                    dsts1_v[pl.ds(16 * j, 16)] = dstv[pl.ds(base1 + 16 * j, 16)]
                h0.wait()
                weight(rows0_v, base0)
                pltpu.sync_copy(rows0_v, sh_num.at[dsts0_v], add=True)
                h1.wait()
                weight(rows1_v, base1)
                pltpu.sync_copy(rows1_v, sh_num.at[dsts1_v], add=True)
                return 0
            lax.fori_loop(0, CH // (2 * K), fpair, 0)
            return 0
        lax.fori_loop(0, ept // CH, chunk, 0)

        @pl.when(cid == 0)
        def _():
            iota = lax.iota(jnp.int32, 16)
            for b in range(DROWS // K):
                for j in range(K // 16):
                    dsts_v[pl.ds(16 * j, 16)] = iota + (b * K + 16 * j)
                pltpu.sync_copy(den_v.at[pl.ds(b * K, K)],
                                sh_den.at[dsts_v], add=True)

        plsc.subcore_barrier()

        pltpu.sync_copy(sh_num.at[pl.ds(sid * srows, srows)],
                        num_h.at[cid, pl.ds(sid * srows, srows)])

        @pl.when(cid == 0)
        def _():
            pltpu.sync_copy(
                sh_den.at[pl.ds(sid * (DROWS // 16), DROWS // 16)],
                den_h.at[pl.ds(sid * (DROWS // 16), DROWS // 16)])

    return pl.kernel(
        body,
        out_type=(jax.ShapeDtypeStruct((2, NPAD, HH), F32),
                  jax.ShapeDtypeStruct((DROWS, 16), F32)),
        mesh=_mesh,
        scratch_types=[
            pltpu.VMEM((CH,), jnp.int32),
            pltpu.VMEM((CH,), jnp.int32),
            pltpu.VMEM((NPAD,), F32),
            pltpu.VMEM((NPAD,), F32),
            pltpu.VMEM((DROWS, 16), F32),
            pltpu.VMEM((CH + 16,), F32),
            pltpu.VMEM((K, HH), F32),
            pltpu.VMEM((K, HH), F32),
            pltpu.VMEM((K,), jnp.int32),
            pltpu.VMEM((K,), jnp.int32),
            pltpu.VMEM((K,), jnp.int32),
            pltpu.VMEM((K,), jnp.int32),
            pltpu.SemaphoreType.DMA,
            pltpu.SemaphoreType.DMA,
            pltpu.VMEM_SHARED((NPAD, HH), F32),
            pltpu.VMEM_SHARED((DROWS, 16), F32),
        ],
        compiler_params=_sc_params,
    )


_sc_320 = _make_sc_scatter(327680)
_sc_160 = _make_sc_scatter(163840)


def _ids_gather_body(xd_h, ids_h, out_h, idx_v, rows_v, sem):
    base = (lax.axis_index("s") * 2 + lax.axis_index("c")) * 64
    pltpu.sync_copy(ids_h.at[pl.ds(base, 64)], idx_v)
    pltpu.async_copy(xd_h.at[idx_v], rows_v, sem).wait()
    pltpu.sync_copy(rows_v, out_h.at[pl.ds(base, 64)])


_ids_gather = pl.kernel(
    _ids_gather_body,
    out_type=jax.ShapeDtypeStruct((2048, H), F32),
    mesh=_mesh,
    scratch_types=[
        pltpu.VMEM((64,), jnp.int32),
        pltpu.VMEM((64, H), F32),
        pltpu.SemaphoreType.DMA,
    ],
    compiler_params=_sc_params,
)


def _dot(a, b):
    return jnp.dot(a, b, preferred_element_type=F32)


def _att(h, att_row):
    return jnp.sum(h * att_row, axis=1, keepdims=True)


def _pre_body(xt_r, xd_r, wtt_r, wdts_r, wdtd_r, wdd_r, wtdd_r,
              astt_r, adtt_r, asdt_r, addt_r, asdd_r, addd_r, adtd_r,
              htt_o, hdt_o, hdd_o, att_o, adt_o, add_o, atd_o):
    xt = xt_r[:]
    xd = xd_r[:]
    z = jnp.zeros((BLK, 6), F32)

    htt = _dot(xt, wtt_r[:])
    htt_o[:] = htt
    a1 = _att(htt, astt_r[:])
    a2 = _att(htt, adtt_r[:])

    hdt = _dot(xd, wdts_r[:])
    hdt_o[:] = hdt
    a3 = _att(hdt, asdt_r[:])
    a4 = _att(_dot(xt, wdtd_r[:]), addt_r[:])

    hdd = _dot(xd, wdd_r[:])
    hdd_o[:] = hdd
    a5 = _att(hdd, asdd_r[:])
    a6 = _att(hdd, addd_r[:])

    a7 = _att(_dot(xd, wtdd_r[:]), adtd_r[:])

    att_o[:] = jnp.concatenate([a1, a2, z], axis=1)
    adt_o[:] = jnp.concatenate([a3, a4, z], axis=1)
    add_o[:] = jnp.concatenate([a5, a6, z], axis=1)
    atd_o[:] = jnp.concatenate([jnp.zeros((BLK, 1), F32), a7,
                                jnp.zeros((BLK, 6), F32)], axis=1)


def _row_spec(w):
    return pl.BlockSpec((BLK, w), lambda i: (i, 0))


def _full_spec(shape):
    return pl.BlockSpec(shape, lambda i: (0,) * len(shape))


_pre = pl.pallas_call(
    _pre_body,
    grid=(NPAD // BLK,),
    in_specs=[_row_spec(H), _row_spec(H)]
    + [_full_spec((H, H))] * 5
    + [_full_spec((1, H))] * 7,
    out_specs=[_row_spec(H)] * 3 + [_row_spec(8)] * 4,
    out_shape=[jax.ShapeDtypeStruct((NPAD, H), F32)] * 3
    + [jax.ShapeDtypeStruct((NPAD, 8), F32)] * 4,
)


def _combine(n0, n1, d, bias):
    return jnp.concatenate([n0, n1], axis=1) / (d + 1e-16) + bias


def _comb_body(n0_r, n1_r, d_r, b_r, out_o):
    out_o[:] = _combine(n0_r[:], n1_r[:], d_r[:], b_r[:])


_comb = pl.pallas_call(
    _comb_body,
    grid=(NPAD // BLK,),
    in_specs=[_row_spec(HH), _row_spec(HH), _row_spec(1),
              _full_spec((1, H))],
    out_specs=_row_spec(H),
    out_shape=jax.ShapeDtypeStruct((NPAD, H), F32),
)


def _comb_avg_body(n0_r, n1_r, d_r, b_r, p_r, out_o):
    x = _combine(n0_r[:], n1_r[:], d_r[:], b_r[:])
    out_o[:] = 0.5 * (x + p_r[:])


_comb_avg = pl.pallas_call(
    _comb_avg_body,
    grid=(NPAD // BLK,),
    in_specs=[_row_spec(HH), _row_spec(HH), _row_spec(1),
              _full_spec((1, H)), _row_spec(H)],
    out_specs=_row_spec(H),
    out_shape=jax.ShapeDtypeStruct((NPAD, H), F32),
)


def _comb_mid_body(n0_r, n1_r, d_r, b_r, p_r, w_r, a_r,
                   xt_o, htd_o, atd_o):
    x = _combine(n0_r[:], n1_r[:], d_r[:], b_r[:])
    x = 0.5 * (x + p_r[:])
    xt_o[:] = x
    htd = _dot(x, w_r[:])
    htd_o[:] = htd
    a = _att(htd, a_r[:])
    atd_o[:] = jnp.concatenate([a, jnp.zeros((BLK, 7), F32)], axis=1)


_comb_mid = pl.pallas_call(
    _comb_mid_body,
    grid=(NPAD // BLK,),
    in_specs=[_row_spec(HH), _row_spec(HH), _row_spec(1),
              _full_spec((1, H)), _row_spec(H),
              _full_spec((H, H)), _full_spec((1, H))],
    out_specs=[_row_spec(H), _row_spec(H), _row_spec(8)],
    out_shape=[jax.ShapeDtypeStruct((NPAD, H), F32),
               jax.ShapeDtypeStruct((NPAD, H), F32),
               jax.ShapeDtypeStruct((NPAD, 8), F32)],
)


def _norm_rows(x):
    n = jnp.sqrt(jnp.sum(x * x, axis=1, keepdims=True))
    return x / jnp.maximum(n, 1e-12)


def _head_body(d1_r, d2_r, cf_r, wc1_r, bc1_r, wc2_r, bc2_r, wc3_r, bc3_r,
               wr1_r, br1_r, wr2_r, br2_r, wr3_r, br3_r, wcl_r, bcl_r,
               out_o):
    x = _norm_rows(cf_r[:])
    x = jnp.maximum(_dot(x, wc1_r[:]) + bc1_r[:], 0.0)
    x = jnp.maximum(_dot(x, wc2_r[:]) + bc2_r[:], 0.0)
    x = jnp.maximum(_dot(x, wc3_r[:]) + bc3_r[:], 0.0)
    h = jnp.concatenate([d1_r[:], d2_r[:], x], axis=1)
    h = _norm_rows(h)
    h = jnp.maximum(_dot(h, wr1_r[:]) + br1_r[:], 0.0)
    h = jnp.maximum(_dot(h, wr2_r[:]) + br2_r[:], 0.0)
    h = jnp.maximum(_dot(h, wr3_r[:]) + br3_r[:], 0.0)
    out_o[:] = _dot(h, wcl_r[:]) + bcl_r[:]


_head = pl.pallas_call(
    _head_body,
    grid=(4,),
    in_specs=[_row_spec(H), _row_spec(H), _row_spec(512),
              _full_spec((512, 512)), _full_spec((1, 512)),
              _full_spec((512, 256)), _full_spec((1, 256)),
              _full_spec((256, 256)), _full_spec((1, 256)),
              _full_spec((512, 512)), _full_spec((1, 512)),
              _full_spec((512, 256)), _full_spec((1, 256)),
              _full_spec((256, H)), _full_spec((1, H)),
              _full_spec((H, H)), _full_spec((1, H))],
    out_specs=_row_spec(H),
    out_shape=jax.ShapeDtypeStruct((1024, H), F32),
)


def _pad_edges(ei, e_pad):
    pe = e_pad - ei.shape[1]
    src = jnp.concatenate([ei[0], jnp.zeros((pe,), jnp.int32)])
    dst = jnp.concatenate([ei[1], jnp.full((pe,), NPAD - 1, jnp.int32)])
    return src, dst


def _conv(sc, ei, e_pad, asrc, adst, h, bias, prior=None, mid=None):
    src, dst = _pad_edges(ei, e_pad)
    h2 = jnp.stack([h[:, :HH], h[:, HH:]])
    num_p, den_p = sc(src, dst, asrc + 0.0, adst + 0.0, h2)
    d = den_p.reshape(NPAD, 1)
    if mid is not None:
        return _comb_mid(num_p[0], num_p[1], d, bias.reshape(1, H),
                         prior, mid[0], mid[1].reshape(1, H))
    if prior is not None:
        return _comb_avg(num_p[0], num_p[1], d, bias.reshape(1, H), prior)
    return _comb(num_p[0], num_p[1], d, bias.reshape(1, H))


def kernel(drug1_id, drug2_id, cell_features, x_target, x_drug,
           ei_tt, ei_dt, ei_dd, ei_td,
           W_tt, att_src_tt, att_dst_tt, b_tt,
           W_dt_src, W_dt_dst, att_src_dt, att_dst_dt, b_dt,
           W_dd, att_src_dd, att_dst_dd, b_dd,
           W_td_src, W_td_dst, att_src_td, att_dst_td, b_td,
           Wc1, bc1, Wc2, bc2, Wc3, bc3,
           Wr1, br1, Wr2, br2, Wr3, br3,
           Wcl, bcl):
    pad = NPAD - N
    xt_p = jnp.pad(x_target, ((0, pad), (0, 0)))
    xd_p = jnp.pad(x_drug, ((0, pad), (0, 0)))
    col = lambda v: v.reshape(1, H)

    htt, hdt, hdd, att_tt, att_dt, att_dd, att_td_d = _pre(
        xt_p, xd_p, W_tt, W_dt_src, W_dt_dst, W_dd, W_td_dst,
        col(att_src_tt), col(att_dst_tt), col(att_src_dt), col(att_dst_dt),
        col(att_src_dd), col(att_dst_dd), col(att_dst_td))

    x_tt = _conv(_sc_320, ei_tt, 327680, att_tt[:, 0], att_tt[:, 1],
                 htt, b_tt)
    xt, htd, att_td_s = _conv(_sc_160, ei_dt, 163840, att_dt[:, 0],
                              att_dt[:, 1], hdt, b_dt, prior=x_tt,
                              mid=(W_td_src, att_src_td))
    x_dd = _conv(_sc_320, ei_dd, 327680, att_dd[:, 0], att_dd[:, 1],
                 hdd, b_dd)
    xd = _conv(_sc_160, ei_td, 163840, att_td_s[:, 0], att_td_d[:, 1],
               htd, b_td, prior=x_dd)

    ids = jnp.concatenate([drug1_id, drug2_id]).astype(jnp.int32)
    d12 = _ids_gather(xd, ids)
    d1 = d12[:1024]
    d2 = d12[1024:]

    row = lambda v, w: v.reshape(1, w)
    wcl_p = jnp.pad(Wcl, ((0, 0), (0, H - 2)))
    bcl_p = jnp.pad(bcl, (0, H - 2)).reshape(1, H)
    out = _head(d1, d2, cell_features,
                Wc1, row(bc1, 512), Wc2, row(bc2, 256), Wc3, row(bc3, 256),
                Wr1, row(br1, 512), Wr2, row(br2, 256), Wr3, row(br3, H),
                wcl_p, bcl_p)

    return (out[:, :2], xt[:N], xd[:N])


# speedup vs baseline: 1.0532x; 1.0532x over previous
import functools

import jax
import jax.numpy as jnp
from jax import lax
from jax.experimental import pallas as pl
from jax.experimental.pallas import tpu as pltpu
from jax.experimental.pallas import tpu_sc as plsc

N = 10000
NPAD = 10240
H = 128
K = 128
BLK = 256
F32 = jnp.float32
DROWS = NPAD // 16

_mesh = plsc.VectorSubcoreMesh(core_axis_name="c", subcore_axis_name="s")
_sc_params = pltpu.CompilerParams(needs_layout_passes=False,
                                  use_tc_tiling_on_sc=False)


HH = H // 2
CH = 2048


def _make_sc_scatter(e_pad):
    ept = e_pad // 16
    assert ept % CH == 0 and CH % K == 0

    def body(src_h, dst_h, asrc_h, adst_h, h_h,
             num_h, den_h,
             srcv, dstv, asrc_v, adst_v, den_v, ex_v,
             rows0_v, rows1_v, srcs0_v, srcs1_v, dsts0_v, dsts1_v,
             sem0, sem1, sem2, sem3,
             sh_num, sh_den):
        rows_v = rows0_v
        dsts_v = dsts0_v
        sid = lax.axis_index("s")
        cid = lax.axis_index("c")
        e0 = sid * ept
        srows = NPAD // 16

        z16 = jnp.zeros((16,), F32)

        def zrow(r, _):
            for j in range(HH // 16):
                rows_v[r, pl.ds(16 * j, 16)] = z16
            return 0
        lax.fori_loop(0, K, zrow, 0)

        def zden(r, _):
            den_v[r] = z16
            return 0
        lax.fori_loop(0, DROWS, zden, 0)

        for b in range(srows // K):
            pltpu.sync_copy(rows_v, sh_num.at[pl.ds(sid * srows + b * K, K)])

        @pl.when(cid == 0)
        def _():
            pltpu.sync_copy(den_v.at[pl.ds(0, DROWS // 16)],
                            sh_den.at[pl.ds(sid * (DROWS // 16),
                                            DROWS // 16)])

        pltpu.sync_copy(asrc_h, asrc_v)
        pltpu.sync_copy(adst_h, adst_v)

        plsc.subcore_barrier()

        def chunk(c, _):
            pltpu.sync_copy(src_h.at[pl.ds(e0 + c * CH, CH)], srcv)
            pltpu.sync_copy(dst_h.at[pl.ds(e0 + c * CH, CH)], dstv)

            def grp(g):
                s = srcv[pl.ds(g * 16, 16)]
                d = dstv[pl.ds(g * 16, 16)]
                av = plsc.load_gather(asrc_v, [s])
                ad = plsc.load_gather(adst_v, [d])
                e = av + ad
                e = jnp.where(e >= 0, e, 0.2 * e)
                ex = jnp.exp(e)
                ex_v[pl.ds(g * 16, 16)] = ex

            def grp_den(g):
                s = srcv[pl.ds(g * 16, 16)]
                d = dstv[pl.ds(g * 16, 16)]
                av = plsc.load_gather(asrc_v, [s])
                ad = plsc.load_gather(adst_v, [d])
                e = av + ad
                e = jnp.where(e >= 0, e, 0.2 * e)
                ex = jnp.exp(e)
                ex_v[pl.ds(g * 16, 16)] = ex
                plsc.addupdate_scatter(
                    den_v, [lax.shift_right_logical(d, 4), d & 15], ex)

            @pl.when(cid == 0)
            def _():
                plsc.parallel_loop(0, CH // 16, 1, unroll=4)(grp_den)

            @pl.when(cid == 1)
            def _():
                plsc.parallel_loop(0, CH // 16, 1, unroll=4)(grp)

            def weight(buf, base):
                @plsc.parallel_loop(0, K, 1, unroll=4)
                def _(r):
                    w = jnp.full((16,), ex_v[pl.ds(base + r, 16)][0])
                    for j in range(HH // 16):
                        sl = pl.ds(16 * j, 16)
                        buf[r, sl] = buf[r, sl] * w

            def fpair(fp, _):
                base0 = fp * 2 * K
                base1 = base0 + K
                for j in range(K // 16):
                    srcs0_v[pl.ds(16 * j, 16)] = srcv[pl.ds(base0 + 16 * j, 16)]
                h0 = pltpu.async_copy(h_h.at[cid].at[srcs0_v], rows0_v, sem0)
                for j in range(K // 16):
                    srcs1_v[pl.ds(16 * j, 16)] = srcv[pl.ds(base1 + 16 * j, 16)]
                h1 = pltpu.async_copy(h_h.at[cid].at[srcs1_v], rows1_v, sem1)
                for j in range(K // 16):
                    dsts0_v[pl.ds(16 * j, 16)] = dstv[pl.ds(base0 + 16 * j, 16)]
                    dsts1_v[pl.ds(16 * j, 16)] = dstv[pl.ds(base1 + 16 * j, 16)]
                h0.wait()
                weight(rows0_v, base0)
                s0 = pltpu.async_copy(rows0_v, sh_num.at[dsts0_v], sem2,
                                      add=True)
                h1.wait()
                weight(rows1_v, base1)
                s1 = pltpu.async_copy(rows1_v, sh_num.at[dsts1_v], sem3,
                                      add=True)
                s0.wait()
                s1.wait()
                return 0
            lax.fori_loop(0, CH // (2 * K), fpair, 0)
            return 0
        lax.fori_loop(0, ept // CH, chunk, 0)

        @pl.when(cid == 0)
        def _():
            iota = lax.iota(jnp.int32, 16)
            for b in range(DROWS // K):
                for j in range(K // 16):
                    dsts_v[pl.ds(16 * j, 16)] = iota + (b * K + 16 * j)
                pltpu.sync_copy(den_v.at[pl.ds(b * K, K)],
                                sh_den.at[dsts_v], add=True)

        plsc.subcore_barrier()

        pltpu.sync_copy(sh_num.at[pl.ds(sid * srows, srows)],
                        num_h.at[cid, pl.ds(sid * srows, srows)])

        @pl.when(cid == 0)
        def _():
            pltpu.sync_copy(
                sh_den.at[pl.ds(sid * (DROWS // 16), DROWS // 16)],
                den_h.at[pl.ds(sid * (DROWS // 16), DROWS // 16)])

    return pl.kernel(
        body,
        out_type=(jax.ShapeDtypeStruct((2, NPAD, HH), F32),
                  jax.ShapeDtypeStruct((DROWS, 16), F32)),
        mesh=_mesh,
        scratch_types=[
            pltpu.VMEM((CH,), jnp.int32),
            pltpu.VMEM((CH,), jnp.int32),
            pltpu.VMEM((NPAD,), F32),
            pltpu.VMEM((NPAD,), F32),
            pltpu.VMEM((DROWS, 16), F32),
            pltpu.VMEM((CH + 16,), F32),
            pltpu.VMEM((K, HH), F32),
            pltpu.VMEM((K, HH), F32),
            pltpu.VMEM((K,), jnp.int32),
            pltpu.VMEM((K,), jnp.int32),
            pltpu.VMEM((K,), jnp.int32),
            pltpu.VMEM((K,), jnp.int32),
            pltpu.SemaphoreType.DMA,
            pltpu.SemaphoreType.DMA,
            pltpu.SemaphoreType.DMA,
            pltpu.SemaphoreType.DMA,
            pltpu.VMEM_SHARED((NPAD, HH), F32),
            pltpu.VMEM_SHARED((DROWS, 16), F32),
        ],
        compiler_params=_sc_params,
    )


_sc_320 = _make_sc_scatter(327680)
_sc_160 = _make_sc_scatter(163840)


def _ids_gather_body(xd_h, ids_h, out_h, idx_v, rows_v, sem):
    base = (lax.axis_index("s") * 2 + lax.axis_index("c")) * 64
    pltpu.sync_copy(ids_h.at[pl.ds(base, 64)], idx_v)
    pltpu.async_copy(xd_h.at[idx_v], rows_v, sem).wait()
    pltpu.sync_copy(rows_v, out_h.at[pl.ds(base, 64)])


_ids_gather = pl.kernel(
    _ids_gather_body,
    out_type=jax.ShapeDtypeStruct((2048, H), F32),
    mesh=_mesh,
    scratch_types=[
        pltpu.VMEM((64,), jnp.int32),
        pltpu.VMEM((64, H), F32),
        pltpu.SemaphoreType.DMA,
    ],
    compiler_params=_sc_params,
)


def _dot(a, b):
    return jnp.dot(a, b, preferred_element_type=F32)


def _att(h, att_row):
    return jnp.sum(h * att_row, axis=1, keepdims=True)


def _pre_body(xt_r, xd_r, wtt_r, wdts_r, wdtd_r, wdd_r, wtdd_r,
              astt_r, adtt_r, asdt_r, addt_r, asdd_r, addd_r, adtd_r,
              htt_o, hdt_o, hdd_o, att_o, adt_o, add_o, atd_o):
    xt = xt_r[:]
    xd = xd_r[:]
    z = jnp.zeros((BLK, 6), F32)

    htt = _dot(xt, wtt_r[:])
    htt_o[:] = htt
    a1 = _att(htt, astt_r[:])
    a2 = _att(htt, adtt_r[:])

    hdt = _dot(xd, wdts_r[:])
    hdt_o[:] = hdt
    a3 = _att(hdt, asdt_r[:])
    a4 = _att(_dot(xt, wdtd_r[:]), addt_r[:])

    hdd = _dot(xd, wdd_r[:])
    hdd_o[:] = hdd
    a5 = _att(hdd, asdd_r[:])
    a6 = _att(hdd, addd_r[:])

    a7 = _att(_dot(xd, wtdd_r[:]), adtd_r[:])

    att_o[:] = jnp.concatenate([a1, a2, z], axis=1)
    adt_o[:] = jnp.concatenate([a3, a4, z], axis=1)
    add_o[:] = jnp.concatenate([a5, a6, z], axis=1)
    atd_o[:] = jnp.concatenate([jnp.zeros((BLK, 1), F32), a7,
                                jnp.zeros((BLK, 6), F32)], axis=1)


def _row_spec(w):
    return pl.BlockSpec((BLK, w), lambda i: (i, 0))


def _full_spec(shape):
    return pl.BlockSpec(shape, lambda i: (0,) * len(shape))


_pre = pl.pallas_call(
    _pre_body,
    grid=(NPAD // BLK,),
    in_specs=[_row_spec(H), _row_spec(H)]
    + [_full_spec((H, H))] * 5
    + [_full_spec((1, H))] * 7,
    out_specs=[_row_spec(H)] * 3 + [_row_spec(8)] * 4,
    out_shape=[jax.ShapeDtypeStruct((NPAD, H), F32)] * 3
    + [jax.ShapeDtypeStruct((NPAD, 8), F32)] * 4,
)


def _combine(n0, n1, d, bias):
    return jnp.concatenate([n0, n1], axis=1) / (d + 1e-16) + bias


def _comb_body(n0_r, n1_r, d_r, b_r, out_o):
    out_o[:] = _combine(n0_r[:], n1_r[:], d_r[:], b_r[:])


_comb = pl.pallas_call(
    _comb_body,
    grid=(NPAD // BLK,),
    in_specs=[_row_spec(HH), _row_spec(HH), _row_spec(1),
              _full_spec((1, H))],
    out_specs=_row_spec(H),
    out_shape=jax.ShapeDtypeStruct((NPAD, H), F32),
)


def _comb_avg_body(n0_r, n1_r, d_r, b_r, p_r, out_o):
    x = _combine(n0_r[:], n1_r[:], d_r[:], b_r[:])
    out_o[:] = 0.5 * (x + p_r[:])


_comb_avg = pl.pallas_call(
    _comb_avg_body,
    grid=(NPAD // BLK,),
    in_specs=[_row_spec(HH), _row_spec(HH), _row_spec(1),
              _full_spec((1, H)), _row_spec(H)],
    out_specs=_row_spec(H),
    out_shape=jax.ShapeDtypeStruct((NPAD, H), F32),
)


def _comb_mid_body(n0_r, n1_r, d_r, b_r, p_r, w_r, a_r,
                   xt_o, htd_o, atd_o):
    x = _combine(n0_r[:], n1_r[:], d_r[:], b_r[:])
    x = 0.5 * (x + p_r[:])
    xt_o[:] = x
    htd = _dot(x, w_r[:])
    htd_o[:] = htd
    a = _att(htd, a_r[:])
    atd_o[:] = jnp.concatenate([a, jnp.zeros((BLK, 7), F32)], axis=1)


_comb_mid = pl.pallas_call(
    _comb_mid_body,
    grid=(NPAD // BLK,),
    in_specs=[_row_spec(HH), _row_spec(HH), _row_spec(1),
              _full_spec((1, H)), _row_spec(H),
              _full_spec((H, H)), _full_spec((1, H))],
    out_specs=[_row_spec(H), _row_spec(H), _row_spec(8)],
    out_shape=[jax.ShapeDtypeStruct((NPAD, H), F32),
               jax.ShapeDtypeStruct((NPAD, H), F32),
               jax.ShapeDtypeStruct((NPAD, 8), F32)],
)


def _norm_rows(x):
    n = jnp.sqrt(jnp.sum(x * x, axis=1, keepdims=True))
    return x / jnp.maximum(n, 1e-12)


def _head_body(d1_r, d2_r, cf_r, wc1_r, bc1_r, wc2_r, bc2_r, wc3_r, bc3_r,
               wr1_r, br1_r, wr2_r, br2_r, wr3_r, br3_r, wcl_r, bcl_r,
               out_o):
    x = _norm_rows(cf_r[:])
    x = jnp.maximum(_dot(x, wc1_r[:]) + bc1_r[:], 0.0)
    x = jnp.maximum(_dot(x, wc2_r[:]) + bc2_r[:], 0.0)
    x = jnp.maximum(_dot(x, wc3_r[:]) + bc3_r[:], 0.0)
    h = jnp.concatenate([d1_r[:], d2_r[:], x], axis=1)
    h = _norm_rows(h)
    h = jnp.maximum(_dot(h, wr1_r[:]) + br1_r[:], 0.0)
    h = jnp.maximum(_dot(h, wr2_r[:]) + br2_r[:], 0.0)
    h = jnp.maximum(_dot(h, wr3_r[:]) + br3_r[:], 0.0)
    out_o[:] = _dot(h, wcl_r[:]) + bcl_r[:]


_head = pl.pallas_call(
    _head_body,
    grid=(4,),
    in_specs=[_row_spec(H), _row_spec(H), _row_spec(512),
              _full_spec((512, 512)), _full_spec((1, 512)),
              _full_spec((512, 256)), _full_spec((1, 256)),
              _full_spec((256, 256)), _full_spec((1, 256)),
              _full_spec((512, 512)), _full_spec((1, 512)),
              _full_spec((512, 256)), _full_spec((1, 256)),
              _full_spec((256, H)), _full_spec((1, H)),
              _full_spec((H, H)), _full_spec((1, H))],
    out_specs=_row_spec(H),
    out_shape=jax.ShapeDtypeStruct((1024, H), F32),
)


def _pad_edges(ei, e_pad):
    pe = e_pad - ei.shape[1]
    src = jnp.concatenate([ei[0], jnp.zeros((pe,), jnp.int32)])
    dst = jnp.concatenate([ei[1], jnp.full((pe,), NPAD - 1, jnp.int32)])
    return src, dst


def _conv(sc, ei, e_pad, asrc, adst, h, bias, prior=None, mid=None):
    src, dst = _pad_edges(ei, e_pad)
    h2 = jnp.stack([h[:, :HH], h[:, HH:]])
    num_p, den_p = sc(src, dst, asrc + 0.0, adst + 0.0, h2)
    d = den_p.reshape(NPAD, 1)
    if mid is not None:
        return _comb_mid(num_p[0], num_p[1], d, bias.reshape(1, H),
                         prior, mid[0], mid[1].reshape(1, H))
    if prior is not None:
        return _comb_avg(num_p[0], num_p[1], d, bias.reshape(1, H), prior)
    return _comb(num_p[0], num_p[1], d, bias.reshape(1, H))


def kernel(drug1_id, drug2_id, cell_features, x_target, x_drug,
           ei_tt, ei_dt, ei_dd, ei_td,
           W_tt, att_src_tt, att_dst_tt, b_tt,
           W_dt_src, W_dt_dst, att_src_dt, att_dst_dt, b_dt,
           W_dd, att_src_dd, att_dst_dd, b_dd,
           W_td_src, W_td_dst, att_src_td, att_dst_td, b_td,
           Wc1, bc1, Wc2, bc2, Wc3, bc3,
           Wr1, br1, Wr2, br2, Wr3, br3,
           Wcl, bcl):
    pad = NPAD - N
    xt_p = jnp.pad(x_target, ((0, pad), (0, 0)))
    xd_p = jnp.pad(x_drug, ((0, pad), (0, 0)))
    col = lambda v: v.reshape(1, H)

    htt, hdt, hdd, att_tt, att_dt, att_dd, att_td_d = _pre(
        xt_p, xd_p, W_tt, W_dt_src, W_dt_dst, W_dd, W_td_dst,
        col(att_src_tt), col(att_dst_tt), col(att_src_dt), col(att_dst_dt),
        col(att_src_dd), col(att_dst_dd), col(att_dst_td))

    x_tt = _conv(_sc_320, ei_tt, 327680, att_tt[:, 0], att_tt[:, 1],
                 htt, b_tt)
    xt, htd, att_td_s = _conv(_sc_160, ei_dt, 163840, att_dt[:, 0],
                              att_dt[:, 1], hdt, b_dt, prior=x_tt,
                              mid=(W_td_src, att_src_td))
    x_dd = _conv(_sc_320, ei_dd, 327680, att_dd[:, 0], att_dd[:, 1],
                 hdd, b_dd)
    xd = _conv(_sc_160, ei_td, 163840, att_td_s[:, 0], att_td_d[:, 1],
               htd, b_td, prior=x_dd)

    ids = jnp.concatenate([drug1_id, drug2_id]).astype(jnp.int32)
    d12 = _ids_gather(xd, ids)
    d1 = d12[:1024]
    d2 = d12[1024:]

    row = lambda v, w: v.reshape(1, w)
    wcl_p = jnp.pad(Wcl, ((0, 0), (0, H - 2)))
    bcl_p = jnp.pad(bcl, (0, H - 2)).reshape(1, H)
    out = _head(d1, d2, cell_features,
                Wc1, row(bc1, 512), Wc2, row(bc2, 256), Wc3, row(bc3, 256),
                Wr1, row(br1, 512), Wr2, row(br2, 256), Wr3, row(br3, H),
                wcl_p, bcl_p)

    return (out[:, :2], xt[:N], xd[:N])


# speedup vs baseline: 1.5351x; 1.4576x over previous
import functools

import jax
import jax.numpy as jnp
from jax import lax
from jax.experimental import pallas as pl
from jax.experimental.pallas import tpu as pltpu
from jax.experimental.pallas import tpu_sc as plsc

N = 10000
NPAD = 10240
H = 128
K = 128
BLK = 256
F32 = jnp.float32
DROWS = NPAD // 16

_mesh = plsc.VectorSubcoreMesh(core_axis_name="c", subcore_axis_name="s")
_sc_params = pltpu.CompilerParams(needs_layout_passes=False,
                                  use_tc_tiling_on_sc=False)


HH = H // 2
CH = 2048


def _make_sc_scatter(e_pad):
    ept = e_pad // 16
    assert ept % CH == 0 and CH % K == 0

    def body(src_h, dst_h, asrc_h, adst_h, h_h,
             num_h, den_h,
             srcv, dstv, asrc_v, adst_v, zb_v, ex_v,
             rows0_v, rows1_v, srcs0_v, srcs1_v, dsts0_v, dsts1_v,
             sem0, sem1, sem2, sem3, sem4,
             sh_num, sh_den, sh_h):
        rows_v = rows0_v
        sid = lax.axis_index("s")
        cid = lax.axis_index("c")
        e0 = sid * ept
        srows = NPAD // 16

        z16 = jnp.zeros((16,), F32)

        def zrow(r, _):
            for j in range(HH // 16):
                rows_v[r, pl.ds(16 * j, 16)] = z16
            return 0
        lax.fori_loop(0, K, zrow, 0)

        def zden(r, _):
            zb_v[pl.ds(16 * r, 16)] = z16
            return 0
        lax.fori_loop(0, srows // 16, zden, 0)

        for b in range(srows // K):
            pltpu.sync_copy(rows_v, sh_num.at[pl.ds(sid * srows + b * K, K)])
        pltpu.sync_copy(zb_v, sh_den.at[pl.ds(sid * srows, srows)])

        pltpu.sync_copy(asrc_h, asrc_v)
        pltpu.sync_copy(adst_h, adst_v)

        pltpu.sync_copy(h_h.at[cid, pl.ds(sid * srows, srows)],
                        sh_h.at[pl.ds(sid * srows, srows)])

        plsc.subcore_barrier()

        def chunk(c, _):
            pltpu.sync_copy(src_h.at[pl.ds(e0 + c * CH, CH)], srcv)
            pltpu.sync_copy(dst_h.at[pl.ds(e0 + c * CH, CH)], dstv)

            def grp(g):
                s = srcv[pl.ds(g * 16, 16)]
                d = dstv[pl.ds(g * 16, 16)]
                av = plsc.load_gather(asrc_v, [s])
                ad = plsc.load_gather(adst_v, [d])
                e = av + ad
                e = jnp.where(e >= 0, e, 0.2 * e)
                ex = jnp.exp(e)
                ex_v[pl.ds(g * 16, 16)] = ex

            plsc.parallel_loop(0, CH // 16, 1, unroll=4)(grp)

            @pl.when(cid == 0)
            def _():
                pltpu.async_copy(ex_v.at[pl.ds(0, CH)], sh_den.at[dstv],
                                 sem4, add=True).wait()

            def weight(buf, base):
                @plsc.parallel_loop(0, K, 1, unroll=4)
                def _(r):
                    w = jnp.full((16,), ex_v[pl.ds(base + r, 16)][0])
                    for j in range(HH // 16):
                        sl = pl.ds(16 * j, 16)
                        buf[r, sl] = buf[r, sl] * w

            def fpair(fp, _):
                base0 = fp * 2 * K
                base1 = base0 + K
                for j in range(K // 16):
                    srcs0_v[pl.ds(16 * j, 16)] = srcv[pl.ds(base0 + 16 * j, 16)]
                h0 = pltpu.async_copy(sh_h.at[srcs0_v], rows0_v, sem0)
                for j in range(K // 16):
                    srcs1_v[pl.ds(16 * j, 16)] = srcv[pl.ds(base1 + 16 * j, 16)]
                h1 = pltpu.async_copy(sh_h.at[srcs1_v], rows1_v, sem1)
                for j in range(K // 16):
                    dsts0_v[pl.ds(16 * j, 16)] = dstv[pl.ds(base0 + 16 * j, 16)]
                    dsts1_v[pl.ds(16 * j, 16)] = dstv[pl.ds(base1 + 16 * j, 16)]
                h0.wait()
                weight(rows0_v, base0)
                s0 = pltpu.async_copy(rows0_v, sh_num.at[dsts0_v], sem2,
                                      add=True)
                h1.wait()
                weight(rows1_v, base1)
                s1 = pltpu.async_copy(rows1_v, sh_num.at[dsts1_v], sem3,
                                      add=True)
                s0.wait()
                s1.wait()
                return 0
            lax.fori_loop(0, CH // (2 * K), fpair, 0)
            return 0
        lax.fori_loop(0, ept // CH, chunk, 0)

        plsc.subcore_barrier()

        pltpu.sync_copy(sh_num.at[pl.ds(sid * srows, srows)],
                        num_h.at[cid, pl.ds(sid * srows, srows)])

        @pl.when(cid == 0)
        def _():
            pltpu.sync_copy(sh_den.at[pl.ds(sid * srows, srows)],
                            den_h.at[pl.ds(sid * srows, srows)])

    return pl.kernel(
        body,
        out_type=(jax.ShapeDtypeStruct((2, NPAD, HH), F32),
                  jax.ShapeDtypeStruct((NPAD,), F32)),
        mesh=_mesh,
        scratch_types=[
            pltpu.VMEM((CH,), jnp.int32),
            pltpu.VMEM((CH,), jnp.int32),
            pltpu.VMEM((NPAD,), F32),
            pltpu.VMEM((NPAD,), F32),
            pltpu.VMEM((NPAD // 16,), F32),
            pltpu.VMEM((CH + 16,), F32),
            pltpu.VMEM((K, HH), F32),
            pltpu.VMEM((K, HH), F32),
            pltpu.VMEM((K,), jnp.int32),
            pltpu.VMEM((K,), jnp.int32),
            pltpu.VMEM((K,), jnp.int32),
            pltpu.VMEM((K,), jnp.int32),
            pltpu.SemaphoreType.DMA,
            pltpu.SemaphoreType.DMA,
            pltpu.SemaphoreType.DMA,
            pltpu.SemaphoreType.DMA,
            pltpu.SemaphoreType.DMA,
            pltpu.VMEM_SHARED((NPAD, HH), F32),
            pltpu.VMEM_SHARED((NPAD,), F32),
            pltpu.VMEM_SHARED((NPAD, HH), F32),
        ],
        compiler_params=_sc_params,
    )


_sc_320 = _make_sc_scatter(327680)
_sc_160 = _make_sc_scatter(163840)


def _ids_gather_body(xd_h, ids_h, out_h, idx_v, rows_v, sem):
    base = (lax.axis_index("s") * 2 + lax.axis_index("c")) * 64
    pltpu.sync_copy(ids_h.at[pl.ds(base, 64)], idx_v)
    pltpu.async_copy(xd_h.at[idx_v], rows_v, sem).wait()
    pltpu.sync_copy(rows_v, out_h.at[pl.ds(base, 64)])


_ids_gather = pl.kernel(
    _ids_gather_body,
    out_type=jax.ShapeDtypeStruct((2048, H), F32),
    mesh=_mesh,
    scratch_types=[
        pltpu.VMEM((64,), jnp.int32),
        pltpu.VMEM((64, H), F32),
        pltpu.SemaphoreType.DMA,
    ],
    compiler_params=_sc_params,
)


def _dot(a, b):
    return jnp.dot(a, b, preferred_element_type=F32)


def _att(h, att_row):
    return jnp.sum(h * att_row, axis=1, keepdims=True)


def _pre_body(xt_r, xd_r, wtt_r, wdts_r, wdtd_r, wdd_r, wtdd_r,
              astt_r, adtt_r, asdt_r, addt_r, asdd_r, addd_r, adtd_r,
              htt_o, hdt_o, hdd_o, att_o, adt_o, add_o, atd_o):
    xt = xt_r[:]
    xd = xd_r[:]
    z = jnp.zeros((BLK, 6), F32)

    htt = _dot(xt, wtt_r[:])
    htt_o[:] = htt
    a1 = _att(htt, astt_r[:])
    a2 = _att(htt, adtt_r[:])

    hdt = _dot(xd, wdts_r[:])
    hdt_o[:] = hdt
    a3 = _att(hdt, asdt_r[:])
    a4 = _att(_dot(xt, wdtd_r[:]), addt_r[:])

    hdd = _dot(xd, wdd_r[:])
    hdd_o[:] = hdd
    a5 = _att(hdd, asdd_r[:])
    a6 = _att(hdd, addd_r[:])

    a7 = _att(_dot(xd, wtdd_r[:]), adtd_r[:])

    att_o[:] = jnp.concatenate([a1, a2, z], axis=1)
    adt_o[:] = jnp.concatenate([a3, a4, z], axis=1)
    add_o[:] = jnp.concatenate([a5, a6, z], axis=1)
    atd_o[:] = jnp.concatenate([jnp.zeros((BLK, 1), F32), a7,
                                jnp.zeros((BLK, 6), F32)], axis=1)


def _row_spec(w):
    return pl.BlockSpec((BLK, w), lambda i: (i, 0))


def _full_spec(shape):
    return pl.BlockSpec(shape, lambda i: (0,) * len(shape))


_pre = pl.pallas_call(
    _pre_body,
    grid=(NPAD // BLK,),
    in_specs=[_row_spec(H), _row_spec(H)]
    + [_full_spec((H, H))] * 5
    + [_full_spec((1, H))] * 7,
    out_specs=[_row_spec(H)] * 3 + [_row_spec(8)] * 4,
    out_shape=[jax.ShapeDtypeStruct((NPAD, H), F32)] * 3
    + [jax.ShapeDtypeStruct((NPAD, 8), F32)] * 4,
)


def _combine(n0, n1, d, bias):
    return jnp.concatenate([n0, n1], axis=1) / (d + 1e-16) + bias


def _comb_body(n0_r, n1_r, d_r, b_r, out_o):
    out_o[:] = _combine(n0_r[:], n1_r[:], d_r[:], b_r[:])


_comb = pl.pallas_call(
    _comb_body,
    grid=(NPAD // BLK,),
    in_specs=[_row_spec(HH), _row_spec(HH), _row_spec(1),
              _full_spec((1, H))],
    out_specs=_row_spec(H),
    out_shape=jax.ShapeDtypeStruct((NPAD, H), F32),
)


def _comb_avg_body(n0_r, n1_r, d_r, b_r, p_r, out_o):
    x = _combine(n0_r[:], n1_r[:], d_r[:], b_r[:])
    out_o[:] = 0.5 * (x + p_r[:])


_comb_avg = pl.pallas_call(
    _comb_avg_body,
    grid=(NPAD // BLK,),
    in_specs=[_row_spec(HH), _row_spec(HH), _row_spec(1),
              _full_spec((1, H)), _row_spec(H)],
    out_specs=_row_spec(H),
    out_shape=jax.ShapeDtypeStruct((NPAD, H), F32),
)


def _comb_mid_body(n0_r, n1_r, d_r, b_r, p_r, w_r, a_r,
                   xt_o, htd_o, atd_o):
    x = _combine(n0_r[:], n1_r[:], d_r[:], b_r[:])
    x = 0.5 * (x + p_r[:])
    xt_o[:] = x
    htd = _dot(x, w_r[:])
    htd_o[:] = htd
    a = _att(htd, a_r[:])
    atd_o[:] = jnp.concatenate([a, jnp.zeros((BLK, 7), F32)], axis=1)


_comb_mid = pl.pallas_call(
    _comb_mid_body,
    grid=(NPAD // BLK,),
    in_specs=[_row_spec(HH), _row_spec(HH), _row_spec(1),
              _full_spec((1, H)), _row_spec(H),
              _full_spec((H, H)), _full_spec((1, H))],
    out_specs=[_row_spec(H), _row_spec(H), _row_spec(8)],
    out_shape=[jax.ShapeDtypeStruct((NPAD, H), F32),
               jax.ShapeDtypeStruct((NPAD, H), F32),
               jax.ShapeDtypeStruct((NPAD, 8), F32)],
)


def _norm_rows(x):
    n = jnp.sqrt(jnp.sum(x * x, axis=1, keepdims=True))
    return x / jnp.maximum(n, 1e-12)


def _head_body(d1_r, d2_r, cf_r, wc1_r, bc1_r, wc2_r, bc2_r, wc3_r, bc3_r,
               wr1_r, br1_r, wr2_r, br2_r, wr3_r, br3_r, wcl_r, bcl_r,
               out_o):
    x = _norm_rows(cf_r[:])
    x = jnp.maximum(_dot(x, wc1_r[:]) + bc1_r[:], 0.0)
    x = jnp.maximum(_dot(x, wc2_r[:]) + bc2_r[:], 0.0)
    x = jnp.maximum(_dot(x, wc3_r[:]) + bc3_r[:], 0.0)
    h = jnp.concatenate([d1_r[:], d2_r[:], x], axis=1)
    h = _norm_rows(h)
    h = jnp.maximum(_dot(h, wr1_r[:]) + br1_r[:], 0.0)
    h = jnp.maximum(_dot(h, wr2_r[:]) + br2_r[:], 0.0)
    h = jnp.maximum(_dot(h, wr3_r[:]) + br3_r[:], 0.0)
    out_o[:] = _dot(h, wcl_r[:]) + bcl_r[:]


_head = pl.pallas_call(
    _head_body,
    grid=(4,),
    in_specs=[_row_spec(H), _row_spec(H), _row_spec(512),
              _full_spec((512, 512)), _full_spec((1, 512)),
              _full_spec((512, 256)), _full_spec((1, 256)),
              _full_spec((256, 256)), _full_spec((1, 256)),
              _full_spec((512, 512)), _full_spec((1, 512)),
              _full_spec((512, 256)), _full_spec((1, 256)),
              _full_spec((256, H)), _full_spec((1, H)),
              _full_spec((H, H)), _full_spec((1, H))],
    out_specs=_row_spec(H),
    out_shape=jax.ShapeDtypeStruct((1024, H), F32),
)


def _pad_edges(ei, e_pad):
    pe = e_pad - ei.shape[1]
    src = jnp.concatenate([ei[0], jnp.zeros((pe,), jnp.int32)])
    dst = jnp.concatenate([ei[1], jnp.full((pe,), NPAD - 1, jnp.int32)])
    return src, dst


def _conv(sc, ei, e_pad, asrc, adst, h, bias, prior=None, mid=None):
    src, dst = _pad_edges(ei, e_pad)
    h2 = jnp.stack([h[:, :HH], h[:, HH:]])
    num_p, den_p = sc(src, dst, asrc + 0.0, adst + 0.0, h2)
    d = den_p.reshape(NPAD, 1)
    if mid is not None:
        return _comb_mid(num_p[0], num_p[1], d, bias.reshape(1, H),
                         prior, mid[0], mid[1].reshape(1, H))
    if prior is not None:
        return _comb_avg(num_p[0], num_p[1], d, bias.reshape(1, H), prior)
    return _comb(num_p[0], num_p[1], d, bias.reshape(1, H))


def kernel(drug1_id, drug2_id, cell_features, x_target, x_drug,
           ei_tt, ei_dt, ei_dd, ei_td,
           W_tt, att_src_tt, att_dst_tt, b_tt,
           W_dt_src, W_dt_dst, att_src_dt, att_dst_dt, b_dt,
           W_dd, att_src_dd, att_dst_dd, b_dd,
           W_td_src, W_td_dst, att_src_td, att_dst_td, b_td,
           Wc1, bc1, Wc2, bc2, Wc3, bc3,
           Wr1, br1, Wr2, br2, Wr3, br3,
           Wcl, bcl):
    pad = NPAD - N
    xt_p = jnp.pad(x_target, ((0, pad), (0, 0)))
    xd_p = jnp.pad(x_drug, ((0, pad), (0, 0)))
    col = lambda v: v.reshape(1, H)

    htt, hdt, hdd, att_tt, att_dt, att_dd, att_td_d = _pre(
        xt_p, xd_p, W_tt, W_dt_src, W_dt_dst, W_dd, W_td_dst,
        col(att_src_tt), col(att_dst_tt), col(att_src_dt), col(att_dst_dt),
        col(att_src_dd), col(att_dst_dd), col(att_dst_td))

    x_tt = _conv(_sc_320, ei_tt, 327680, att_tt[:, 0], att_tt[:, 1],
                 htt, b_tt)
    xt, htd, att_td_s = _conv(_sc_160, ei_dt, 163840, att_dt[:, 0],
                              att_dt[:, 1], hdt, b_dt, prior=x_tt,
                              mid=(W_td_src, att_src_td))
    x_dd = _conv(_sc_320, ei_dd, 327680, att_dd[:, 0], att_dd[:, 1],
                 hdd, b_dd)
    xd = _conv(_sc_160, ei_td, 163840, att_td_s[:, 0], att_td_d[:, 1],
               htd, b_td, prior=x_dd)

    ids = jnp.concatenate([drug1_id, drug2_id]).astype(jnp.int32)
    d12 = _ids_gather(xd, ids)
    d1 = d12[:1024]
    d2 = d12[1024:]

    row = lambda v, w: v.reshape(1, w)
    wcl_p = jnp.pad(Wcl, ((0, 0), (0, H - 2)))
    bcl_p = jnp.pad(bcl, (0, H - 2)).reshape(1, H)
    out = _head(d1, d2, cell_features,
                Wc1, row(bc1, 512), Wc2, row(bc2, 256), Wc3, row(bc3, 256),
                Wr1, row(br1, 512), Wr2, row(br2, 256), Wr3, row(br3, H),
                wcl_p, bcl_p)

    return (out[:, :2], xt[:N], xd[:N])


# speedup vs baseline: 1.7117x; 1.1150x over previous
import functools

import jax
import jax.numpy as jnp
from jax import lax
from jax.experimental import pallas as pl
from jax.experimental.pallas import tpu as pltpu
from jax.experimental.pallas import tpu_sc as plsc

N = 10000
NPAD = 10240
H = 128
K = 128
BLK = 256
F32 = jnp.float32
DROWS = NPAD // 16

_mesh = plsc.VectorSubcoreMesh(core_axis_name="c", subcore_axis_name="s")
_sc_params = pltpu.CompilerParams(needs_layout_passes=False,
                                  use_tc_tiling_on_sc=False)


HH = H // 2
CH = 2048


def _make_sc_scatter(e_pad):
    ept = e_pad // 16
    assert ept % CH == 0 and CH % K == 0

    def body(src_h, dst_h, asrc_h, adst_h, h_h,
             num_h, den_h,
             srcv, dstv, asrc_v, adst_v, zb_v, ex_v,
             rows0_v, rows1_v, srcs0_v, srcs1_v, dsts0_v, dsts1_v,
             sem0, sem1, sem2, sem3, sem4,
             sh_num, sh_den, sh_h):
        rows_v = rows0_v
        sid = lax.axis_index("s")
        cid = lax.axis_index("c")
        e0 = sid * ept
        srows = NPAD // 16

        z16 = jnp.zeros((16,), F32)

        def zrow(r, _):
            for j in range(HH // 16):
                rows_v[r, pl.ds(16 * j, 16)] = z16
            return 0
        lax.fori_loop(0, K, zrow, 0)

        def zden(r, _):
            zb_v[pl.ds(16 * r, 16)] = z16
            return 0
        lax.fori_loop(0, srows // 16, zden, 0)

        for b in range(srows // K):
            pltpu.sync_copy(rows_v, sh_num.at[pl.ds(sid * srows + b * K, K)])
        pltpu.sync_copy(zb_v, sh_den.at[pl.ds(sid * srows, srows)])

        pltpu.sync_copy(asrc_h, asrc_v)
        pltpu.sync_copy(adst_h, adst_v)

        pltpu.sync_copy(h_h.at[cid, pl.ds(sid * srows, srows)],
                        sh_h.at[pl.ds(sid * srows, srows)])

        plsc.subcore_barrier()

        def chunk(c, _):
            pltpu.sync_copy(src_h.at[pl.ds(e0 + c * CH, CH)], srcv)
            pltpu.sync_copy(dst_h.at[pl.ds(e0 + c * CH, CH)], dstv)

            def grp(g):
                s = srcv[pl.ds(g * 16, 16)]
                d = dstv[pl.ds(g * 16, 16)]
                av = plsc.load_gather(asrc_v, [s])
                ad = plsc.load_gather(adst_v, [d])
                e = av + ad
                e = jnp.where(e >= 0, e, 0.2 * e)
                ex = jnp.exp(e)
                ex_v[pl.ds(g * 16, 16)] = ex

            plsc.parallel_loop(0, CH // 16, 1, unroll=4)(grp)

            @pl.when(cid == 0)
            def _():
                pltpu.async_copy(ex_v.at[pl.ds(0, CH)], sh_den.at[dstv],
                                 sem4, add=True).wait()

            def weight(buf, base):
                @plsc.parallel_loop(0, K, 1, unroll=4)
                def _(r):
                    w = jnp.full((16,), ex_v[pl.ds(base + r, 16)][0])
                    for j in range(HH // 16):
                        sl = pl.ds(16 * j, 16)
                        buf[r, sl] = buf[r, sl] * w

            nblk = CH // K
            sidx = [srcs0_v, srcs1_v]
            didx = [dsts0_v, dsts1_v]
            bufs = [rows0_v, rows1_v]
            gsem = [sem0, sem1]
            ssem = [sem2, sem3]
            pg = [None, None]
            ps = [None, None]
            for i in range(nblk + 1):
                if i < nblk:
                    b = i & 1
                    if ps[b] is not None:
                        ps[b].wait()
                    base = i * K
                    for j in range(K // 16):
                        sidx[b][pl.ds(16 * j, 16)] = (
                            srcv[pl.ds(base + 16 * j, 16)])
                        didx[b][pl.ds(16 * j, 16)] = (
                            dstv[pl.ds(base + 16 * j, 16)])
                    pg[b] = pltpu.async_copy(sh_h.at[sidx[b]], bufs[b],
                                             gsem[b])
                if i >= 1:
                    b = (i - 1) & 1
                    pg[b].wait()
                    weight(bufs[b], (i - 1) * K)
                    ps[b] = pltpu.async_copy(bufs[b], sh_num.at[didx[b]],
                                             ssem[b], add=True)
            ps[0].wait()
            ps[1].wait()
            return 0
        lax.fori_loop(0, ept // CH, chunk, 0)

        plsc.subcore_barrier()

        pltpu.sync_copy(sh_num.at[pl.ds(sid * srows, srows)],
                        num_h.at[cid, pl.ds(sid * srows, srows)])

        @pl.when(cid == 0)
        def _():
            pltpu.sync_copy(sh_den.at[pl.ds(sid * srows, srows)],
                            den_h.at[pl.ds(sid * srows, srows)])

    return pl.kernel(
        body,
        out_type=(jax.ShapeDtypeStruct((2, NPAD, HH), F32),
                  jax.ShapeDtypeStruct((NPAD,), F32)),
        mesh=_mesh,
        scratch_types=[
            pltpu.VMEM((CH,), jnp.int32),
            pltpu.VMEM((CH,), jnp.int32),
            pltpu.VMEM((NPAD,), F32),
            pltpu.VMEM((NPAD,), F32),
            pltpu.VMEM((NPAD // 16,), F32),
            pltpu.VMEM((CH + 16,), F32),
            pltpu.VMEM((K, HH), F32),
            pltpu.VMEM((K, HH), F32),
            pltpu.VMEM((K,), jnp.int32),
            pltpu.VMEM((K,), jnp.int32),
            pltpu.VMEM((K,), jnp.int32),
            pltpu.VMEM((K,), jnp.int32),
            pltpu.SemaphoreType.DMA,
            pltpu.SemaphoreType.DMA,
            pltpu.SemaphoreType.DMA,
            pltpu.SemaphoreType.DMA,
            pltpu.SemaphoreType.DMA,
            pltpu.VMEM_SHARED((NPAD, HH), F32),
            pltpu.VMEM_SHARED((NPAD,), F32),
            pltpu.VMEM_SHARED((NPAD, HH), F32),
        ],
        compiler_params=_sc_params,
    )


_sc_320 = _make_sc_scatter(327680)
_sc_160 = _make_sc_scatter(163840)


def _ids_gather_body(xd_h, ids_h, out_h, idx_v, rows_v, sem):
    base = (lax.axis_index("s") * 2 + lax.axis_index("c")) * 64
    pltpu.sync_copy(ids_h.at[pl.ds(base, 64)], idx_v)
    pltpu.async_copy(xd_h.at[idx_v], rows_v, sem).wait()
    pltpu.sync_copy(rows_v, out_h.at[pl.ds(base, 64)])


_ids_gather = pl.kernel(
    _ids_gather_body,
    out_type=jax.ShapeDtypeStruct((2048, H), F32),
    mesh=_mesh,
    scratch_types=[
        pltpu.VMEM((64,), jnp.int32),
        pltpu.VMEM((64, H), F32),
        pltpu.SemaphoreType.DMA,
    ],
    compiler_params=_sc_params,
)


def _dot(a, b):
    return jnp.dot(a, b, preferred_element_type=F32)


def _att(h, att_row):
    return jnp.sum(h * att_row, axis=1, keepdims=True)


def _pre_body(xt_r, xd_r, wtt_r, wdts_r, wdtd_r, wdd_r, wtdd_r,
              astt_r, adtt_r, asdt_r, addt_r, asdd_r, addd_r, adtd_r,
              htt_o, hdt_o, hdd_o, att_o, adt_o, add_o, atd_o):
    xt = xt_r[:]
    xd = xd_r[:]
    z = jnp.zeros((BLK, 6), F32)

    htt = _dot(xt, wtt_r[:])
    htt_o[:] = htt
    a1 = _att(htt, astt_r[:])
    a2 = _att(htt, adtt_r[:])

    hdt = _dot(xd, wdts_r[:])
    hdt_o[:] = hdt
    a3 = _att(hdt, asdt_r[:])
    a4 = _att(_dot(xt, wdtd_r[:]), addt_r[:])

    hdd = _dot(xd, wdd_r[:])
    hdd_o[:] = hdd
    a5 = _att(hdd, asdd_r[:])
    a6 = _att(hdd, addd_r[:])

    a7 = _att(_dot(xd, wtdd_r[:]), adtd_r[:])

    att_o[:] = jnp.concatenate([a1, a2, z], axis=1)
    adt_o[:] = jnp.concatenate([a3, a4, z], axis=1)
    add_o[:] = jnp.concatenate([a5, a6, z], axis=1)
    atd_o[:] = jnp.concatenate([jnp.zeros((BLK, 1), F32), a7,
                                jnp.zeros((BLK, 6), F32)], axis=1)


def _row_spec(w):
    return pl.BlockSpec((BLK, w), lambda i: (i, 0))


def _full_spec(shape):
    return pl.BlockSpec(shape, lambda i: (0,) * len(shape))


_pre = pl.pallas_call(
    _pre_body,
    grid=(NPAD // BLK,),
    in_specs=[_row_spec(H), _row_spec(H)]
    + [_full_spec((H, H))] * 5
    + [_full_spec((1, H))] * 7,
    out_specs=[_row_spec(H)] * 3 + [_row_spec(8)] * 4,
    out_shape=[jax.ShapeDtypeStruct((NPAD, H), F32)] * 3
    + [jax.ShapeDtypeStruct((NPAD, 8), F32)] * 4,
)


def _combine(n0, n1, d, bias):
    return jnp.concatenate([n0, n1], axis=1) / (d + 1e-16) + bias


def _comb_body(n0_r, n1_r, d_r, b_r, out_o):
    out_o[:] = _combine(n0_r[:], n1_r[:], d_r[:], b_r[:])


_comb = pl.pallas_call(
    _comb_body,
    grid=(NPAD // BLK,),
    in_specs=[_row_spec(HH), _row_spec(HH), _row_spec(1),
              _full_spec((1, H))],
    out_specs=_row_spec(H),
    out_shape=jax.ShapeDtypeStruct((NPAD, H), F32),
)


def _comb_avg_body(n0_r, n1_r, d_r, b_r, p_r, out_o):
    x = _combine(n0_r[:], n1_r[:], d_r[:], b_r[:])
    out_o[:] = 0.5 * (x + p_r[:])


_comb_avg = pl.pallas_call(
    _comb_avg_body,
    grid=(NPAD // BLK,),
    in_specs=[_row_spec(HH), _row_spec(HH), _row_spec(1),
              _full_spec((1, H)), _row_spec(H)],
    out_specs=_row_spec(H),
    out_shape=jax.ShapeDtypeStruct((NPAD, H), F32),
)


def _comb_mid_body(n0_r, n1_r, d_r, b_r, p_r, w_r, a_r,
                   xt_o, htd_o, atd_o):
    x = _combine(n0_r[:], n1_r[:], d_r[:], b_r[:])
    x = 0.5 * (x + p_r[:])
    xt_o[:] = x
    htd = _dot(x, w_r[:])
    htd_o[:] = htd
    a = _att(htd, a_r[:])
    atd_o[:] = jnp.concatenate([a, jnp.zeros((BLK, 7), F32)], axis=1)


_comb_mid = pl.pallas_call(
    _comb_mid_body,
    grid=(NPAD // BLK,),
    in_specs=[_row_spec(HH), _row_spec(HH), _row_spec(1),
              _full_spec((1, H)), _row_spec(H),
              _full_spec((H, H)), _full_spec((1, H))],
    out_specs=[_row_spec(H), _row_spec(H), _row_spec(8)],
    out_shape=[jax.ShapeDtypeStruct((NPAD, H), F32),
               jax.ShapeDtypeStruct((NPAD, H), F32),
               jax.ShapeDtypeStruct((NPAD, 8), F32)],
)


def _norm_rows(x):
    n = jnp.sqrt(jnp.sum(x * x, axis=1, keepdims=True))
    return x / jnp.maximum(n, 1e-12)


def _head_body(d1_r, d2_r, cf_r, wc1_r, bc1_r, wc2_r, bc2_r, wc3_r, bc3_r,
               wr1_r, br1_r, wr2_r, br2_r, wr3_r, br3_r, wcl_r, bcl_r,
               out_o):
    x = _norm_rows(cf_r[:])
    x = jnp.maximum(_dot(x, wc1_r[:]) + bc1_r[:], 0.0)
    x = jnp.maximum(_dot(x, wc2_r[:]) + bc2_r[:], 0.0)
    x = jnp.maximum(_dot(x, wc3_r[:]) + bc3_r[:], 0.0)
    h = jnp.concatenate([d1_r[:], d2_r[:], x], axis=1)
    h = _norm_rows(h)
    h = jnp.maximum(_dot(h, wr1_r[:]) + br1_r[:], 0.0)
    h = jnp.maximum(_dot(h, wr2_r[:]) + br2_r[:], 0.0)
    h = jnp.maximum(_dot(h, wr3_r[:]) + br3_r[:], 0.0)
    out_o[:] = _dot(h, wcl_r[:]) + bcl_r[:]


_head = pl.pallas_call(
    _head_body,
    grid=(4,),
    in_specs=[_row_spec(H), _row_spec(H), _row_spec(512),
              _full_spec((512, 512)), _full_spec((1, 512)),
              _full_spec((512, 256)), _full_spec((1, 256)),
              _full_spec((256, 256)), _full_spec((1, 256)),
              _full_spec((512, 512)), _full_spec((1, 512)),
              _full_spec((512, 256)), _full_spec((1, 256)),
              _full_spec((256, H)), _full_spec((1, H)),
              _full_spec((H, H)), _full_spec((1, H))],
    out_specs=_row_spec(H),
    out_shape=jax.ShapeDtypeStruct((1024, H), F32),
)


def _pad_edges(ei, e_pad):
    pe = e_pad - ei.shape[1]
    src = jnp.concatenate([ei[0], jnp.zeros((pe,), jnp.int32)])
    dst = jnp.concatenate([ei[1], jnp.full((pe,), NPAD - 1, jnp.int32)])
    return src, dst


def _conv(sc, ei, e_pad, asrc, adst, h, bias, prior=None, mid=None):
    src, dst = _pad_edges(ei, e_pad)
    h2 = jnp.stack([h[:, :HH], h[:, HH:]])
    num_p, den_p = sc(src, dst, asrc + 0.0, adst + 0.0, h2)
    d = den_p.reshape(NPAD, 1)
    if mid is not None:
        return _comb_mid(num_p[0], num_p[1], d, bias.reshape(1, H),
                         prior, mid[0], mid[1].reshape(1, H))
    if prior is not None:
        return _comb_avg(num_p[0], num_p[1], d, bias.reshape(1, H), prior)
    return _comb(num_p[0], num_p[1], d, bias.reshape(1, H))


def kernel(drug1_id, drug2_id, cell_features, x_target, x_drug,
           ei_tt, ei_dt, ei_dd, ei_td,
           W_tt, att_src_tt, att_dst_tt, b_tt,
           W_dt_src, W_dt_dst, att_src_dt, att_dst_dt, b_dt,
           W_dd, att_src_dd, att_dst_dd, b_dd,
           W_td_src, W_td_dst, att_src_td, att_dst_td, b_td,
           Wc1, bc1, Wc2, bc2, Wc3, bc3,
           Wr1, br1, Wr2, br2, Wr3, br3,
           Wcl, bcl):
    pad = NPAD - N
    xt_p = jnp.pad(x_target, ((0, pad), (0, 0)))
    xd_p = jnp.pad(x_drug, ((0, pad), (0, 0)))
    col = lambda v: v.reshape(1, H)

    htt, hdt, hdd, att_tt, att_dt, att_dd, att_td_d = _pre(
        xt_p, xd_p, W_tt, W_dt_src, W_dt_dst, W_dd, W_td_dst,
        col(att_src_tt), col(att_dst_tt), col(att_src_dt), col(att_dst_dt),
        col(att_src_dd), col(att_dst_dd), col(att_dst_td))

    x_tt = _conv(_sc_320, ei_tt, 327680, att_tt[:, 0], att_tt[:, 1],
                 htt, b_tt)
    xt, htd, att_td_s = _conv(_sc_160, ei_dt, 163840, att_dt[:, 0],
                              att_dt[:, 1], hdt, b_dt, prior=x_tt,
                              mid=(W_td_src, att_src_td))
    x_dd = _conv(_sc_320, ei_dd, 327680, att_dd[:, 0], att_dd[:, 1],
                 hdd, b_dd)
    xd = _conv(_sc_160, ei_td, 163840, att_td_s[:, 0], att_td_d[:, 1],
               htd, b_td, prior=x_dd)

    ids = jnp.concatenate([drug1_id, drug2_id]).astype(jnp.int32)
    d12 = _ids_gather(xd, ids)
    d1 = d12[:1024]
    d2 = d12[1024:]

    row = lambda v, w: v.reshape(1, w)
    wcl_p = jnp.pad(Wcl, ((0, 0), (0, H - 2)))
    bcl_p = jnp.pad(bcl, (0, H - 2)).reshape(1, H)
    out = _head(d1, d2, cell_features,
                Wc1, row(bc1, 512), Wc2, row(bc2, 256), Wc3, row(bc3, 256),
                Wr1, row(br1, 512), Wr2, row(br2, 256), Wr3, row(br3, H),
                wcl_p, bcl_p)

    return (out[:, :2], xt[:N], xd[:N])
